# Initial kernel scaffold; baseline (speedup 1.0000x reference)
import jax
import jax.numpy as jnp
from jax.experimental import pallas as pl


def kernel(xs, tables):
    raise NotImplementedError("write your pallas kernel here")


# speedup vs baseline: 1.1025x; 1.1025x over previous
import functools
import math

import jax
import jax.numpy as jnp
from jax import lax
from jax.experimental import pallas as pl
from jax.experimental.pallas import tpu as pltpu
from jax.experimental.pallas import tpu_sc as plsc

_F = 26
_V = 100000
_E = 16
_B = 16384
_ROWS = _B * _F
_SCALE = math.sqrt(_F * _V * _E)

_NC = 2
_NS = 16
_NW = _NC * _NS
_RPW = _ROWS // _NW

_CH = 128
_KCH = 8
_SROWS = _CH * _KCH
_NSUP = _RPW // _SROWS
_NCHUNK = _RPW // _CH
_NBUF = 4


def _sc_body(xs_hbm, tab_hbm, out_hbm,
             idx_v, rb0, rb1, rb2, rb3,
             gs0, gs1, gs2, gs3, os0, os1, os2, os3):
    rows = (rb0, rb1, rb2, rb3)
    gsems = (gs0, gs1, gs2, gs3)
    osems = (os0, os1, os2, os3)

    c = lax.axis_index("c")
    s_ = lax.axis_index("s")
    w = s_ * _NC + c
    base = w * _RPW

    pltpu.sync_copy(xs_hbm.at[w], idx_v)

    lanes = lax.iota(jnp.int32, 16)

    def _prep(s):
        def body_k(k, carry):
            for m in range(_CH // 16):
                sl = pl.ds(m * 16, 16)
                p = (base + k * _CH + m * 16) + lanes
                f = lax.rem(p, _F)
                idx_v[k, sl] = idx_v[k, sl] + f * _V
            return carry
        lax.fori_loop(s * _KCH, (s + 1) * _KCH, body_k, 0)

    def _fire(s):
        b = s % _NBUF
        descs = []
        for j in range(_KCH):
            d = pltpu.make_async_copy(
                tab_hbm.at[idx_v.at[s * _KCH + j]],
                rows[b].at[pl.ds(j * _CH, _CH)],
                gsems[b])
            d.start()
            descs.append(d)
        return descs

    def _scale(s):
        b = s % _NBUF
        rb = rows[b]
        unroll = 8
        def body_r(i, carry):
            r0 = i * unroll
            for u in range(unroll):
                rb[r0 + u] = rb[r0 + u] * _SCALE
            return carry
        lax.fori_loop(0, _SROWS // unroll, body_r, 0)

    def _fire_out(s):
        b = s % _NBUF
        d = pltpu.make_async_copy(
            rows[b],
            out_hbm.at[pl.ds(base + s * _SROWS, _SROWS)],
            osems[b])
        d.start()
        return d

    gd = {}
    outcs = {}
    _prep(0)
    gd[0] = _fire(0)
    _prep(1)
    gd[1] = _fire(1)
    for s in range(_NSUP):
        if s + 2 < _NSUP:
            if s - 2 >= 0:
                outcs[s - 2].wait()
            _prep(s + 2)
            gd[s + 2] = _fire(s + 2)
        for d in gd[s]:
            d.wait()
        _scale(s)
        outcs[s] = _fire_out(s)
    for t in range(_NSUP - 4, _NSUP):
        outcs[t].wait()


@jax.jit
def _run(xs_r, tab):
    mesh = plsc.VectorSubcoreMesh(
        core_axis_name="c", subcore_axis_name="s",
        num_cores=_NC, num_subcores=_NS)
    k = functools.partial(
        pl.kernel,
        out_type=jax.ShapeDtypeStruct((_ROWS, _E), jnp.float32),
        mesh=mesh,
        scratch_types=[
            pltpu.VMEM((_NCHUNK, _CH), jnp.int32),
        ] + [pltpu.VMEM((_SROWS, _E), jnp.float32)] * _NBUF
          + [pltpu.SemaphoreType.DMA] * (2 * _NBUF),
        compiler_params=pltpu.CompilerParams(use_tc_tiling_on_sc=False),
    )(_sc_body)
    return k(xs_r, tab)


def kernel(xs, tables):
    xs_r = xs.reshape(_NW, _NCHUNK, _CH)
    tab = tables.reshape(_F * _V, _E)
    out = _run(xs_r, tab)
    return out.reshape(_B, _F * _E)


# speedup vs baseline: 7.2600x; 6.5849x over previous
import functools
import math

import jax
import jax.numpy as jnp
from jax import lax
from jax.experimental import pallas as pl
from jax.experimental.pallas import tpu as pltpu
from jax.experimental.pallas import tpu_sc as plsc

_F = 26
_V = 100000
_E = 16
_B = 16384
_NCOL = _F * _E
_SCALE = math.sqrt(_F * _V * _E)

_NC = 2
_NS = 16
_NW = _NC * _NS
_CPW = _NCOL // _NW

_CB = 2048
_NCHUNK = _B // _CB


def _sc_body(xs_hbm, tab_hbm, out_hbm,
             col_v, idx0, idx1, out0, out1,
             csem, isem, osem):
    idx_v = (idx0, idx1)
    out_v = (out0, out1)

    c_ax = lax.axis_index("c")
    s_ax = lax.axis_index("s")
    w = s_ax * _NC + c_ax

    def do_column(j):
        c = w * _CPW + j
        f = c // _E
        e = lax.rem(c, _E)

        col_dma = pltpu.make_async_copy(tab_hbm.at[f, e], col_v, csem)
        col_dma.start()

        i_dma = pltpu.make_async_copy(xs_hbm.at[f, pl.ds(0, _CB)], idx_v[0], isem)
        i_dma.start()
        col_dma.wait()
        i_dma.wait()

        for t in range(_NCHUNK):
            bsel = t % 2
            if t + 1 < _NCHUNK:
                pltpu.make_async_copy(
                    xs_hbm.at[f, pl.ds((t + 1) * _CB, _CB)],
                    idx_v[1 - bsel], isem).start()
            if t >= 2:
                pltpu.make_async_copy(
                    out_v[bsel], out_hbm.at[c, pl.ds((t - 2) * _CB, _CB)],
                    osem).wait()
            ib = idx_v[bsel]
            ob = out_v[bsel]

            def gather16(i, carry):
                sl = pl.ds(i * 16, 16)
                vals = plsc.load_gather(col_v, [ib[sl]])
                ob[sl] = vals * _SCALE
                return carry
            lax.fori_loop(0, _CB // 16, gather16, 0)

            pltpu.make_async_copy(
                ob, out_hbm.at[c, pl.ds(t * _CB, _CB)], osem).start()
            if t + 1 < _NCHUNK:
                pltpu.make_async_copy(
                    xs_hbm.at[f, pl.ds((t + 1) * _CB, _CB)],
                    idx_v[1 - bsel], isem).wait()

        for t in (_NCHUNK - 2, _NCHUNK - 1):
            pltpu.make_async_copy(
                out_v[t % 2], out_hbm.at[c, pl.ds(t * _CB, _CB)], osem).wait()

    for j in range(_CPW):
        do_column(j)


@jax.jit
def _run(xs_t, tab_t):
    mesh = plsc.VectorSubcoreMesh(
        core_axis_name="c", subcore_axis_name="s",
        num_cores=_NC, num_subcores=_NS)
    k = functools.partial(
        pl.kernel,
        out_type=jax.ShapeDtypeStruct((_NCOL, _B), jnp.float32),
        mesh=mesh,
        scratch_types=[
            pltpu.VMEM((_V,), jnp.float32),
            pltpu.VMEM((_CB,), jnp.int32),
            pltpu.VMEM((_CB,), jnp.int32),
            pltpu.VMEM((_CB,), jnp.float32),
            pltpu.VMEM((_CB,), jnp.float32),
            pltpu.SemaphoreType.DMA,
            pltpu.SemaphoreType.DMA,
            pltpu.SemaphoreType.DMA,
        ],
        compiler_params=pltpu.CompilerParams(needs_layout_passes=False),
    )(_sc_body)
    return k(xs_t, tab_t)


def kernel(xs, tables):
    xs_t = jnp.transpose(xs, (1, 0))
    tab_t = jnp.transpose(tables, (0, 2, 1))
    out_t = _run(xs_t, tab_t)
    return jnp.transpose(out_t, (1, 0))
